# baseline (speedup 1.0000x reference)
import functools

import jax
import jax.numpy as jnp
from jax.experimental import pallas as pl


def _linear_kernel(x_ref, w_ref, b_ref, o_ref):
    o_ref[...] = (
        jnp.dot(x_ref[...], w_ref[...], preferred_element_type=jnp.float32)
        + b_ref[...]
    )


@functools.partial(jax.jit, static_argnames=("block_rows",))
def _linear(x, w, b, block_rows):
    n, k = x.shape
    latent = w.shape[1]
    grid = (n // block_rows,)
    return pl.pallas_call(
        _linear_kernel,
        grid=grid,
        in_specs=[
            pl.BlockSpec((block_rows, k), lambda i: (i, 0)),
            pl.BlockSpec((k, latent), lambda i: (0, 0)),
            pl.BlockSpec((1, latent), lambda i: (0, 0)),
        ],
        out_specs=pl.BlockSpec((block_rows, latent), lambda i: (i, 0)),
        out_shape=jax.ShapeDtypeStruct((n, latent), jnp.float32),
    )(x, w, b.reshape(1, latent))


def kernel(x, edge_attr, W_node, b_node, W_edge, b_edge):
    x_emb = _linear(x, W_node, b_node, block_rows=2000)
    edge_emb = _linear(edge_attr, W_edge, b_edge, block_rows=6400)
    return (x_emb, edge_emb)


# speedup vs baseline: 1.0466x; 1.0466x over previous
import functools

import jax
import jax.numpy as jnp
from jax.experimental import pallas as pl


def _linear_kernel(x_ref, w_ref, b_ref, o_ref):
    o_ref[...] = (
        jnp.dot(x_ref[...], w_ref[...], preferred_element_type=jnp.float32)
        + b_ref[...]
    )


@functools.partial(jax.jit, static_argnames=("block_rows",))
def _linear(x, w, b, block_rows):
    n, k = x.shape
    latent = w.shape[1]
    grid = (n // block_rows,)
    return pl.pallas_call(
        _linear_kernel,
        grid=grid,
        in_specs=[
            pl.BlockSpec((block_rows, k), lambda i: (i, 0)),
            pl.BlockSpec((k, latent), lambda i: (0, 0)),
            pl.BlockSpec((1, latent), lambda i: (0, 0)),
        ],
        out_specs=pl.BlockSpec((block_rows, latent), lambda i: (i, 0)),
        out_shape=jax.ShapeDtypeStruct((n, latent), jnp.float32),
    )(x, w, b.reshape(1, latent))


def kernel(x, edge_attr, W_node, b_node, W_edge, b_edge):
    x_emb = _linear(x, W_node, b_node, block_rows=10000)
    edge_emb = _linear(edge_attr, W_edge, b_edge, block_rows=20000)
    return (x_emb, edge_emb)


# speedup vs baseline: 1.0484x; 1.0017x over previous
import functools

import jax
import jax.numpy as jnp
from jax import lax
from jax.experimental import pallas as pl
from jax.experimental.pallas import tpu as pltpu


def _linear_dma_kernel(x_hbm, w_ref, b_ref, o_hbm, in_buf, out_buf, sem_in,
                       sem_out, *, block_rows, nin, nout):
    n = x_hbm.shape[0]
    nblk = n // block_rows

    def in_copy(i):
        return pltpu.make_async_copy(
            x_hbm.at[pl.ds(i * block_rows, block_rows), :],
            in_buf.at[lax.rem(i, nin)],
            sem_in.at[lax.rem(i, nin)],
        )

    def out_copy(i):
        return pltpu.make_async_copy(
            out_buf.at[lax.rem(i, nout)],
            o_hbm.at[pl.ds(i * block_rows, block_rows), :],
            sem_out.at[lax.rem(i, nout)],
        )

    for k in range(min(nin, nblk)):
        in_copy(k).start()

    def body(i, carry):
        in_copy(i).wait()

        @pl.when(i >= nout)
        def _():
            out_copy(i - nout).wait()

        out_buf[lax.rem(i, nout)] = (
            jnp.dot(in_buf[lax.rem(i, nin)], w_ref[...],
                    preferred_element_type=jnp.float32)
            + b_ref[...]
        )
        out_copy(i).start()

        @pl.when(i + nin < nblk)
        def _():
            in_copy(i + nin).start()

        return carry

    lax.fori_loop(0, nblk, body, 0)

    for k in range(max(nblk - nout, 0), nblk):
        out_copy(k).wait()


@functools.partial(jax.jit, static_argnames=("block_rows", "nin", "nout"))
def _linear(x, w, b, block_rows, nin, nout):
    n, k = x.shape
    latent = w.shape[1]
    return pl.pallas_call(
        functools.partial(_linear_dma_kernel, block_rows=block_rows,
                          nin=nin, nout=nout),
        in_specs=[
            pl.BlockSpec(memory_space=pl.ANY),
            pl.BlockSpec(memory_space=pltpu.VMEM),
            pl.BlockSpec(memory_space=pltpu.VMEM),
        ],
        out_specs=pl.BlockSpec(memory_space=pl.ANY),
        out_shape=jax.ShapeDtypeStruct((n, latent), jnp.float32),
        scratch_shapes=[
            pltpu.VMEM((nin, block_rows, k), jnp.float32),
            pltpu.VMEM((nout, block_rows, latent), jnp.float32),
            pltpu.SemaphoreType.DMA((nin,)),
            pltpu.SemaphoreType.DMA((nout,)),
        ],
    )(x, w, b.reshape(1, latent))


def kernel(x, edge_attr, W_node, b_node, W_edge, b_edge):
    x_emb = _linear(x, W_node, b_node, block_rows=5000, nin=4, nout=6)
    edge_emb = _linear(edge_attr, W_edge, b_edge, block_rows=6400, nin=4, nout=6)
    return (x_emb, edge_emb)
